# baseline (speedup 1.0000x reference)
import functools

import jax
import jax.numpy as jnp
from jax import lax
from jax.experimental import pallas as pl
from jax.experimental.pallas import tpu as pltpu
from jax.experimental.pallas import tpu_sc as plsc

_B, _T, _D, _E, _C = 4096, 16, 1024, 8, 256
_BS = 256
_BS2 = 512
_B_SC = 1024
_B_TC = _B - _B_SC

_NC, _NS, _L = 2, 16, 16
_NW = _NC * _NS


def _fused_body(sid_ref, x_ref, w_ref, b_ref, o_ref):
    xp = jnp.mean(x_ref[...], axis=1)
    sid = sid_ref[0, 0, :]
    acc = jnp.zeros((xp.shape[0], _C), jnp.float32)
    for e in range(_E):
        mask = (sid == e).astype(jnp.float32)[:, None]
        y = jax.lax.dot_general(
            xp, w_ref[e],
            dimension_numbers=(((1,), (1,)), ((), ())),
            preferred_element_type=jnp.float32,
        )
        acc = acc + mask * (y + b_ref[e][None, :])
    o_ref[...] = acc


def _mm_body(sid_ref, xp_ref, w_ref, b_ref, o_ref):
    xp = xp_ref[...]
    sid = sid_ref[0, 0, :]
    acc = jnp.zeros((xp.shape[0], _C), jnp.float32)
    for e in range(_E):
        mask = (sid == e).astype(jnp.float32)[:, None]
        y = jax.lax.dot_general(
            xp, w_ref[e],
            dimension_numbers=(((1,), (1,)), ((), ())),
            preferred_element_type=jnp.float32,
        )
        acc = acc + mask * (y + b_ref[e][None, :])
    o_ref[...] = acc


def _tc_fused(x, sid3, W, b, bs, nb):
    return pl.pallas_call(
        _fused_body,
        grid=(nb,),
        in_specs=[
            pl.BlockSpec((1, 1, bs), lambda i: (i, 0, 0)),
            pl.BlockSpec((bs, _T, _D), lambda i: (i, 0, 0)),
            pl.BlockSpec((_E, _C, _D), lambda i: (0, 0, 0)),
            pl.BlockSpec((_E, _C), lambda i: (0, 0)),
        ],
        out_specs=pl.BlockSpec((bs, _C), lambda i: (i, 0)),
        out_shape=jax.ShapeDtypeStruct((bs * nb, _C), jnp.float32),
    )(sid3, x, W, b)


def _tc_matmul(xp, sid3, W, b, bs, nb):
    return pl.pallas_call(
        _mm_body,
        grid=(nb,),
        in_specs=[
            pl.BlockSpec((1, 1, bs), lambda i: (i, 0, 0)),
            pl.BlockSpec((bs, _D), lambda i: (i, 0)),
            pl.BlockSpec((_E, _C, _D), lambda i: (0, 0, 0)),
            pl.BlockSpec((_E, _C), lambda i: (0, 0)),
        ],
        out_specs=pl.BlockSpec((bs, _C), lambda i: (i, 0)),
        out_shape=jax.ShapeDtypeStruct((bs * nb, _C), jnp.float32),
    )(sid3, xp, W, b)


def _make_sc_pool(b_sc):
    rows_w = b_sc // _NW
    mesh = plsc.VectorSubcoreMesh(core_axis_name="c", subcore_axis_name="s")

    @functools.partial(
        pl.kernel, mesh=mesh,
        out_type=jax.ShapeDtypeStruct((b_sc, _D), jnp.float32),
        scratch_types=[
            pltpu.VMEM((_T, _D), jnp.float32),
            pltpu.VMEM((_T, _D), jnp.float32),
            pltpu.VMEM((_D,), jnp.float32),
            pltpu.SemaphoreType.DMA,
            pltpu.SemaphoreType.DMA,
        ],
    )
    def sc_pool(x_hbm, o_hbm, buf0, buf1, ov, sem0, sem1):
        wid = lax.axis_index("s") * _NC + lax.axis_index("c")
        base = wid * rows_w
        last = base + rows_w - 1

        pltpu.async_copy(x_hbm.at[base], buf0, sem0)
        pltpu.async_copy(x_hbm.at[jnp.minimum(base + 1, last)], buf1, sem1)

        def pool_one(buf, row):
            def chunk(c, carry):
                o = pl.ds(c * _L, _L)
                acc = buf[0, o]
                for t in range(1, _T):
                    acc = acc + buf[t, o]
                ov[o] = acc * (1.0 / _T)
                return carry
            lax.fori_loop(0, _D // _L, chunk, 0)
            pltpu.sync_copy(ov, o_hbm.at[row])

        def step(k, carry):
            r0 = base + 2 * k
            pltpu.make_async_copy(x_hbm.at[r0], buf0, sem0).wait()
            pool_one(buf0, r0)
            pltpu.async_copy(x_hbm.at[jnp.minimum(r0 + 2, last)], buf0, sem0)
            r1 = r0 + 1
            pltpu.make_async_copy(x_hbm.at[r1], buf1, sem1).wait()
            pool_one(buf1, r1)
            pltpu.async_copy(x_hbm.at[jnp.minimum(r1 + 2, last)], buf1, sem1)
            return carry

        lax.fori_loop(0, rows_w // 2, step, 0)
        pltpu.make_async_copy(x_hbm.at[last], buf0, sem0).wait()
        pltpu.make_async_copy(x_hbm.at[last], buf1, sem1).wait()

    return sc_pool


_sc_pool = _make_sc_pool(_B_SC)


@jax.jit
def kernel(x, system_id, W, b):
    sid = system_id.astype(jnp.int32)

    nb1 = _B_TC // _BS
    sid3_tc = sid[:_B_TC].reshape(nb1, 1, _BS)
    out_tc = _tc_fused(x[:_B_TC], sid3_tc, W, b, _BS, nb1)

    xp_sc = _sc_pool(x[_B_TC:])

    nb2 = _B_SC // _BS2
    sid3_sc = sid[_B_TC:].reshape(nb2, 1, _BS2)
    out_sc = _tc_matmul(xp_sc, sid3_sc, W, b, _BS2, nb2)

    return jnp.concatenate([out_tc, out_sc], axis=0)


# speedup vs baseline: 2.1845x; 2.1845x over previous
import functools

import jax
import jax.numpy as jnp
from jax import lax
from jax.experimental import pallas as pl
from jax.experimental.pallas import tpu as pltpu
from jax.experimental.pallas import tpu_sc as plsc

_B, _T, _D, _E, _C = 4096, 16, 1024, 8, 256
_BS = 256
_BS2 = 512
_B_SC = 1024
_B_TC = _B - _B_SC

_NC, _NS, _L = 2, 16, 16
_NW = _NC * _NS


def _fused_body(sid_ref, x_ref, w_ref, b_ref, o_ref):
    xp = jnp.mean(x_ref[...], axis=1)
    sid = sid_ref[0, 0, :]
    acc = jnp.zeros((xp.shape[0], _C), jnp.float32)
    for e in range(_E):
        mask = (sid == e).astype(jnp.float32)[:, None]
        y = jax.lax.dot_general(
            xp, w_ref[e],
            dimension_numbers=(((1,), (1,)), ((), ())),
            preferred_element_type=jnp.float32,
        )
        acc = acc + mask * (y + b_ref[e][None, :])
    o_ref[...] = acc


def _mm_body(sid_ref, xp_ref, w_ref, b_ref, o_ref):
    xp = xp_ref[...]
    sid = sid_ref[0, 0, :]
    acc = jnp.zeros((xp.shape[0], _C), jnp.float32)
    for e in range(_E):
        mask = (sid == e).astype(jnp.float32)[:, None]
        y = jax.lax.dot_general(
            xp, w_ref[e],
            dimension_numbers=(((1,), (1,)), ((), ())),
            preferred_element_type=jnp.float32,
        )
        acc = acc + mask * (y + b_ref[e][None, :])
    o_ref[...] = acc


def _tc_fused(x, sid3, W, b, bs, nb):
    return pl.pallas_call(
        _fused_body,
        grid=(nb,),
        in_specs=[
            pl.BlockSpec((1, 1, bs), lambda i: (i, 0, 0)),
            pl.BlockSpec((bs, _T, _D), lambda i: (i, 0, 0)),
            pl.BlockSpec((_E, _C, _D), lambda i: (0, 0, 0)),
            pl.BlockSpec((_E, _C), lambda i: (0, 0)),
        ],
        out_specs=pl.BlockSpec((bs, _C), lambda i: (i, 0)),
        out_shape=jax.ShapeDtypeStruct((bs * nb, _C), jnp.float32),
    )(sid3, x, W, b)


def _tc_matmul(xp, sid3, W, b, bs, nb, blk_off):
    return pl.pallas_call(
        _mm_body,
        grid=(nb,),
        in_specs=[
            pl.BlockSpec((1, 1, bs), lambda i: (i + blk_off, 0, 0)),
            pl.BlockSpec((bs, _D), lambda i: (i, 0)),
            pl.BlockSpec((_E, _C, _D), lambda i: (0, 0, 0)),
            pl.BlockSpec((_E, _C), lambda i: (0, 0)),
        ],
        out_specs=pl.BlockSpec((bs, _C), lambda i: (i, 0)),
        out_shape=jax.ShapeDtypeStruct((bs * nb, _C), jnp.float32),
    )(sid3, xp, W, b)


def _make_sc_pool(b_sc, row_off):
    rows_w = b_sc // _NW
    mesh = plsc.VectorSubcoreMesh(core_axis_name="c", subcore_axis_name="s")

    @functools.partial(
        pl.kernel, mesh=mesh,
        out_type=jax.ShapeDtypeStruct((b_sc, _D), jnp.float32),
        scratch_types=[
            pltpu.VMEM((_T, _D), jnp.float32),
            pltpu.VMEM((_T, _D), jnp.float32),
            pltpu.VMEM((_D,), jnp.float32),
            pltpu.SemaphoreType.DMA,
            pltpu.SemaphoreType.DMA,
        ],
    )
    def sc_pool(x_hbm, o_hbm, buf0, buf1, ov, sem0, sem1):
        wid = lax.axis_index("s") * _NC + lax.axis_index("c")
        base = wid * rows_w
        last = base + rows_w - 1

        pltpu.async_copy(x_hbm.at[row_off + base], buf0, sem0)
        pltpu.async_copy(x_hbm.at[row_off + jnp.minimum(base + 1, last)],
                         buf1, sem1)

        def pool_one(buf, row):
            for c in range(_D // _L):
                o = pl.ds(c * _L, _L)
                acc = buf[0, o]
                for t in range(1, _T):
                    acc = acc + buf[t, o]
                ov[o] = acc * (1.0 / _T)
            pltpu.sync_copy(ov, o_hbm.at[row])

        def step(k, carry):
            r0 = base + 2 * k
            pltpu.make_async_copy(x_hbm.at[row_off + r0], buf0, sem0).wait()
            pool_one(buf0, r0)
            pltpu.async_copy(x_hbm.at[row_off + jnp.minimum(r0 + 2, last)],
                             buf0, sem0)
            r1 = r0 + 1
            pltpu.make_async_copy(x_hbm.at[row_off + r1], buf1, sem1).wait()
            pool_one(buf1, r1)
            pltpu.async_copy(x_hbm.at[row_off + jnp.minimum(r1 + 2, last)],
                             buf1, sem1)
            return carry

        lax.fori_loop(0, rows_w // 2, step, 0)
        pltpu.make_async_copy(x_hbm.at[row_off + last], buf0, sem0).wait()
        pltpu.make_async_copy(x_hbm.at[row_off + last], buf1, sem1).wait()

    return sc_pool


_sc_pool = _make_sc_pool(_B_SC, _B_TC)


@jax.jit
def kernel(x, system_id, W, b):
    sid = system_id.astype(jnp.int32)

    xp_sc = _sc_pool(x)

    nb1 = _B_TC // _BS
    sid3a = sid.reshape(_B // _BS, 1, _BS)
    out_tc = _tc_fused(x, sid3a, W, b, _BS, nb1)

    nb2 = _B_SC // _BS2
    sid3b = sid.reshape(_B // _BS2, 1, _BS2)
    out_sc = _tc_matmul(xp_sc, sid3b, W, b, _BS2, nb2, _B_TC // _BS2)

    return jnp.concatenate([out_tc, out_sc], axis=0)


# speedup vs baseline: 2.3981x; 1.0978x over previous
import functools

import jax
import jax.numpy as jnp
from jax import lax
from jax.experimental import pallas as pl
from jax.experimental.pallas import tpu as pltpu
from jax.experimental.pallas import tpu_sc as plsc

_B, _T, _D, _E, _C = 4096, 16, 1024, 8, 256
_BS = 256
_BS2 = 512
_B_SC = 1024
_B_TC = _B - _B_SC

_NC, _NS, _L = 2, 16, 16
_NW = _NC * _NS


def _fused_body(sid_ref, x_ref, w_ref, b_ref, o_ref):
    xp = jnp.mean(x_ref[...], axis=1)
    sid = sid_ref[0, 0, :]
    acc = jnp.zeros((xp.shape[0], _C), jnp.float32)
    for e in range(_E):
        mask = (sid == e).astype(jnp.float32)[:, None]
        y = jax.lax.dot_general(
            xp, w_ref[e],
            dimension_numbers=(((1,), (1,)), ((), ())),
            preferred_element_type=jnp.float32,
        )
        acc = acc + mask * (y + b_ref[e][None, :])
    o_ref[...] = acc


def _mm_body(sid_ref, xp_ref, w_ref, b_ref, o_ref):
    xp = xp_ref[...]
    sid = sid_ref[0, 0, :]
    acc = jnp.zeros((xp.shape[0], _C), jnp.float32)
    for e in range(_E):
        mask = (sid == e).astype(jnp.float32)[:, None]
        y = jax.lax.dot_general(
            xp, w_ref[e],
            dimension_numbers=(((1,), (1,)), ((), ())),
            preferred_element_type=jnp.float32,
        )
        acc = acc + mask * (y + b_ref[e][None, :])
    o_ref[...] = acc


def _tc_fused(x, sid3, W, b, bs, nb):
    return pl.pallas_call(
        _fused_body,
        grid=(nb,),
        in_specs=[
            pl.BlockSpec((1, 1, bs), lambda i: (i, 0, 0)),
            pl.BlockSpec((bs, _T, _D), lambda i: (i, 0, 0)),
            pl.BlockSpec((_E, _C, _D), lambda i: (0, 0, 0)),
            pl.BlockSpec((_E, _C), lambda i: (0, 0)),
        ],
        out_specs=pl.BlockSpec((bs, _C), lambda i: (i, 0)),
        out_shape=jax.ShapeDtypeStruct((bs * nb, _C), jnp.float32),
    )(sid3, x, W, b)


def _tc_matmul(xp, sid3, W, b, bs, nb, blk_off):
    return pl.pallas_call(
        _mm_body,
        grid=(nb,),
        in_specs=[
            pl.BlockSpec((1, 1, bs), lambda i: (i + blk_off, 0, 0)),
            pl.BlockSpec((bs, _D), lambda i: (i, 0)),
            pl.BlockSpec((_E, _C, _D), lambda i: (0, 0, 0)),
            pl.BlockSpec((_E, _C), lambda i: (0, 0)),
        ],
        out_specs=pl.BlockSpec((bs, _C), lambda i: (i, 0)),
        out_shape=jax.ShapeDtypeStruct((bs * nb, _C), jnp.float32),
    )(sid3, xp, W, b)


def _make_sc_pool(b_sc, row_off):
    rows_w = b_sc // _NW
    mesh = plsc.VectorSubcoreMesh(core_axis_name="c", subcore_axis_name="s")

    @functools.partial(
        pl.kernel, mesh=mesh,
        out_type=jax.ShapeDtypeStruct((b_sc, _D), jnp.float32),
        scratch_types=[
            pltpu.VMEM((_T, _D), jnp.float32),
            pltpu.VMEM((_T, _D), jnp.float32),
            pltpu.VMEM((_T, _D), jnp.float32),
            pltpu.VMEM((_T, _D), jnp.float32),
            pltpu.VMEM((rows_w, _D), jnp.float32),
            pltpu.SemaphoreType.DMA,
            pltpu.SemaphoreType.DMA,
            pltpu.SemaphoreType.DMA,
            pltpu.SemaphoreType.DMA,
        ],
    )
    def sc_pool(x_hbm, o_hbm, b0, b1, b2, b3, ov, s0, s1, s2, s3):
        bufs = (b0, b1, b2, b3)
        sems = (s0, s1, s2, s3)
        wid = lax.axis_index("s") * _NC + lax.axis_index("c")
        base = wid * rows_w
        last = rows_w - 1

        for j in range(3):
            pltpu.async_copy(x_hbm.at[row_off + base + j], bufs[j], sems[j])

        def pool_one(buf, r_local):
            def chunk(c, carry):
                o = pl.ds(c * _L, _L)
                acc = buf[0, o]
                for t in range(1, _T):
                    acc = acc + buf[t, o]
                ov[r_local, o] = acc * (1.0 / _T)
                return carry
            lax.fori_loop(0, _D // _L, chunk, 0, unroll=8)

        def step(k, carry):
            for j in range(4):
                r = 4 * k + j
                pltpu.make_async_copy(
                    x_hbm.at[row_off + base + r], bufs[j], sems[j]).wait()
                pool_one(bufs[j], r)
                nxt = jnp.minimum(r + 3, last)
                pltpu.async_copy(x_hbm.at[row_off + base + nxt],
                                 bufs[(j + 3) % 4], sems[(j + 3) % 4])
            return carry

        lax.fori_loop(0, rows_w // 4, step, 0)
        for j in range(3):
            pltpu.make_async_copy(
                x_hbm.at[row_off + base + last], bufs[j], sems[j]).wait()
        pltpu.sync_copy(ov, o_hbm.at[pl.ds(base, rows_w)])

    return sc_pool


_sc_pool = _make_sc_pool(_B_SC, _B_TC)


@jax.jit
def kernel(x, system_id, W, b):
    sid = system_id.astype(jnp.int32)

    xp_sc = _sc_pool(x)

    nb1 = _B_TC // _BS
    sid3a = sid.reshape(_B // _BS, 1, _BS)
    out_tc = _tc_fused(x, sid3a, W, b, _BS, nb1)

    nb2 = _B_SC // _BS2
    sid3b = sid.reshape(_B // _BS2, 1, _BS2)
    out_sc = _tc_matmul(xp_sc, sid3b, W, b, _BS2, nb2, _B_TC // _BS2)

    return jnp.concatenate([out_tc, out_sc], axis=0)


# speedup vs baseline: 3.2297x; 1.3468x over previous
import functools

import jax
import jax.numpy as jnp
from jax import lax
from jax.experimental import pallas as pl
from jax.experimental.pallas import tpu as pltpu
from jax.experimental.pallas import tpu_sc as plsc

_B, _T, _D, _E, _C = 4096, 16, 1024, 8, 256
_BS = 256
_BS2 = 512
_B_SC = 1024
_B_TC = _B - _B_SC

_NC, _NS, _L = 2, 16, 16
_NW = _NC * _NS


def _fused_body(sid_ref, x_ref, w_ref, b_ref, o_ref):
    xp = jnp.mean(x_ref[...], axis=1)
    sid = sid_ref[0, 0, :]
    acc = jnp.zeros((xp.shape[0], _C), jnp.float32)
    for e in range(_E):
        mask = (sid == e).astype(jnp.float32)[:, None]
        y = jax.lax.dot_general(
            xp, w_ref[e],
            dimension_numbers=(((1,), (1,)), ((), ())),
            preferred_element_type=jnp.float32,
        )
        acc = acc + mask * (y + b_ref[e][None, :])
    o_ref[...] = acc


def _mm_body(sid_ref, xp_ref, w_ref, b_ref, o_ref):
    xp = xp_ref[...]
    sid = sid_ref[0, 0, :]
    acc = jnp.zeros((xp.shape[0], _C), jnp.float32)
    for e in range(_E):
        mask = (sid == e).astype(jnp.float32)[:, None]
        y = jax.lax.dot_general(
            xp, w_ref[e],
            dimension_numbers=(((1,), (1,)), ((), ())),
            preferred_element_type=jnp.float32,
        )
        acc = acc + mask * (y + b_ref[e][None, :])
    o_ref[...] = acc


def _tc_fused(x, sid3, W, b, bs, nb):
    return pl.pallas_call(
        _fused_body,
        grid=(nb,),
        in_specs=[
            pl.BlockSpec((1, 1, bs), lambda i: (i, 0, 0)),
            pl.BlockSpec((bs, _T, _D), lambda i: (i, 0, 0)),
            pl.BlockSpec((_E, _C, _D), lambda i: (0, 0, 0)),
            pl.BlockSpec((_E, _C), lambda i: (0, 0)),
        ],
        out_specs=pl.BlockSpec((bs, _C), lambda i: (i, 0)),
        out_shape=jax.ShapeDtypeStruct((bs * nb, _C), jnp.float32),
    )(sid3, x, W, b)


def _tc_matmul(xp, sid3, W, b, bs, nb, blk_off):
    return pl.pallas_call(
        _mm_body,
        grid=(nb,),
        in_specs=[
            pl.BlockSpec((1, 1, bs), lambda i: (i + blk_off, 0, 0)),
            pl.BlockSpec((bs, _D), lambda i: (i, 0)),
            pl.BlockSpec((_E, _C, _D), lambda i: (0, 0, 0)),
            pl.BlockSpec((_E, _C), lambda i: (0, 0)),
        ],
        out_specs=pl.BlockSpec((bs, _C), lambda i: (i, 0)),
        out_shape=jax.ShapeDtypeStruct((bs * nb, _C), jnp.float32),
    )(sid3, xp, W, b)


def _make_sc_pool(b_sc, row_off):
    rows_w = b_sc // _NW
    mesh = plsc.VectorSubcoreMesh(core_axis_name="c", subcore_axis_name="s")

    @functools.partial(
        pl.kernel, mesh=mesh,
        out_type=jax.ShapeDtypeStruct((b_sc, _D), jnp.float32),
        scratch_types=[
            pltpu.VMEM((_T, _D), jnp.float32),
            pltpu.VMEM((_T, _D), jnp.float32),
            pltpu.VMEM((_T, _D), jnp.float32),
            pltpu.VMEM((_T, _D), jnp.float32),
            pltpu.VMEM((rows_w, _D), jnp.float32),
            pltpu.SemaphoreType.DMA,
            pltpu.SemaphoreType.DMA,
            pltpu.SemaphoreType.DMA,
            pltpu.SemaphoreType.DMA,
        ],
    )
    def sc_pool(x_hbm, o_hbm, b0, b1, b2, b3, ov, s0, s1, s2, s3):
        bufs = (b0, b1, b2, b3)
        sems = (s0, s1, s2, s3)
        wid = lax.axis_index("s") * _NC + lax.axis_index("c")
        base = wid * rows_w
        last = rows_w - 1

        for j in range(3):
            pltpu.async_copy(x_hbm.at[row_off + base + j], bufs[j], sems[j])

        def pool_one(buf, r_local):
            def chunk(c, carry):
                o = pl.ds(c * _L, _L)
                acc = buf[0, o]
                for t in range(1, _T):
                    acc = acc + buf[t, o]
                ov[r_local, o] = acc * (1.0 / _T)
                return carry
            lax.fori_loop(0, _D // _L, chunk, 0, unroll=8)

        def step(k, carry):
            for j in range(4):
                r = 4 * k + j
                pltpu.make_async_copy(
                    x_hbm.at[row_off + base + r], bufs[j], sems[j]).wait()
                pool_one(bufs[j], r)
                nxt = jnp.minimum(r + 3, last)
                pltpu.async_copy(x_hbm.at[row_off + base + nxt],
                                 bufs[(j + 3) % 4], sems[(j + 3) % 4])
            return carry

        lax.fori_loop(0, rows_w // 4, step, 0)
        for j in range(3):
            pltpu.make_async_copy(
                x_hbm.at[row_off + base + last], bufs[j], sems[j]).wait()
        pltpu.sync_copy(ov, o_hbm.at[pl.ds(base, rows_w)])

    return sc_pool


_sc_pool = _make_sc_pool(_B_SC, _B_TC)


def _dual_body(sid_ref, xa_ref, xb_ref, w_ref, b_ref, o_ref):
    h = xa_ref.shape[0]
    xpa = jnp.mean(xa_ref[...], axis=1)
    xpb = jnp.mean(xb_ref[...], axis=1)
    xp = jnp.concatenate([xpa, xpb], axis=0)
    sid = sid_ref[0, 0, :]
    acc = jnp.zeros((2 * h, _C), jnp.float32)
    for e in range(_E):
        mask = (sid == e).astype(jnp.float32)[:, None]
        y = jax.lax.dot_general(
            xp, w_ref[e],
            dimension_numbers=(((1,), (1,)), ((), ())),
            preferred_element_type=jnp.float32,
        )
        acc = acc + mask * (y + b_ref[e][None, :])
    o_ref[...] = acc


@jax.jit
def kernel(x, system_id, W, b):
    h = 128
    nb = _B // (2 * h)
    sid3 = system_id.astype(jnp.int32).reshape(nb, 1, 2 * h)
    return pl.pallas_call(
        _dual_body,
        grid=(nb,),
        in_specs=[
            pl.BlockSpec((1, 1, 2 * h), lambda i: (i, 0, 0)),
            pl.BlockSpec((h, _T, _D), lambda i: (2 * i, 0, 0)),
            pl.BlockSpec((h, _T, _D), lambda i: (2 * i + 1, 0, 0)),
            pl.BlockSpec((_E, _C, _D), lambda i: (0, 0, 0)),
            pl.BlockSpec((_E, _C), lambda i: (0, 0)),
        ],
        out_specs=pl.BlockSpec((2 * h, _C), lambda i: (i, 0)),
        out_shape=jax.ShapeDtypeStruct((_B, _C), jnp.float32),
    )(sid3, x, x, W, b)


@jax.jit
def _kernel_split(x, system_id, W, b):
    sid = system_id.astype(jnp.int32)

    xp_sc = _sc_pool(x)

    nb1 = _B_TC // _BS
    sid3a = sid.reshape(_B // _BS, 1, _BS)
    out_tc = _tc_fused(x, sid3a, W, b, _BS, nb1)

    nb2 = _B_SC // _BS2
    sid3b = sid.reshape(_B // _BS2, 1, _BS2)
    out_sc = _tc_matmul(xp_sc, sid3b, W, b, _BS2, nb2, _B_TC // _BS2)

    return jnp.concatenate([out_tc, out_sc], axis=0)
